# baseline (speedup 1.0000x reference)
import functools

import jax
import jax.numpy as jnp
from jax import lax
from jax.experimental import pallas as pl
from jax.experimental.pallas import tpu as pltpu
from jax.experimental.pallas import tpu_sc as plsc

VOCAB = 1000000
B = 1024
D = 64
H = 128
L = 20

NC = 2
NS = 16
NW = NC * NS
B_PER_W = B // NW
IDX_PER_W = B_PER_W * L
N_CHUNKS = IDX_PER_W // 128
NBUF = 16
W = 16
D_VECS = D // 16


def _gather_mean_body(idx_hbm, tableT_hbm, out_hbm,
                      idx_v, rows_v, out_v,
                      *sems):
    wid = lax.axis_index("s") * NC + lax.axis_index("c")

    pltpu.sync_copy(idx_hbm.at[wid], idx_v)

    n_steps = IDX_PER_W // NBUF

    def load_chunk(g):
        return idx_v[g // 8, pl.ds((g % 8) * NBUF, NBUF)]

    def fire(i, k):
        off = pl.multiple_of((i // W) * W, W)
        pltpu.async_copy(
            tableT_hbm.at[:, pl.ds(off, W)], rows_v.at[k], sems[k])

    chunk0 = load_chunk(0)
    for k in range(NBUF):
        fire(chunk0[k], k)

    zero = jnp.zeros((16,), jnp.float32)
    inv_l = 1.0 / L

    def step(g, carry):
        accs = list(carry[:D_VECS])
        chunk = carry[D_VECS]
        nxt = load_chunk(jnp.minimum(g + 1, n_steps - 1))
        for k in range(NBUF):
            j = g * NBUF + k
            pltpu.make_async_copy(
                tableT_hbm.at[:, pl.ds(0, W)], rows_v.at[k], sems[k]
            ).wait()
            i = chunk[k]
            lane = jnp.full((16,), i % W, dtype=jnp.int32)
            for g2 in range(D_VECS):
                rows = jnp.arange(16, dtype=jnp.int32) + 16 * g2
                accs[g2] = accs[g2] + plsc.load_gather(
                    rows_v.at[k], [rows, lane])

            @pl.when(g < n_steps - 1)
            def _():
                fire(nxt[k], k)

            b = j // L
            for g2 in range(D_VECS):
                out_v[b, pl.ds(16 * g2, 16)] = accs[g2] * inv_l
            keep = 1.0 - ((j % L) == (L - 1)).astype(jnp.float32)
            for g2 in range(D_VECS):
                accs[g2] = accs[g2] * keep
        return tuple(accs) + (nxt,)

    lax.fori_loop(0, n_steps, step, (zero,) * D_VECS + (chunk0,))

    pltpu.sync_copy(out_v, out_hbm.at[pl.ds(wid * B_PER_W, B_PER_W)])


@functools.cache
def _gather_mean():
    return pl.kernel(
        _gather_mean_body,
        out_type=jax.ShapeDtypeStruct((B, D), jnp.float32),
        mesh=plsc.VectorSubcoreMesh(core_axis_name="c", subcore_axis_name="s"),
        scratch_types=[
            pltpu.VMEM((N_CHUNKS, 128), jnp.int32),
            pltpu.VMEM((NBUF, D, W), jnp.float32),
            pltpu.VMEM((B_PER_W, D), jnp.float32),
        ] + [pltpu.SemaphoreType.DMA] * NBUF,
        compiler_params=pltpu.CompilerParams(
            use_tc_tiling_on_sc=False, needs_layout_passes=False),
    )


def _dense_body(x_ref, w_ih_ref, b_ih_ref, b_hh_ref, w_out_ref, b_out_ref,
                out_ref):
    x = x_ref[...]
    gi = lax.dot_general(
        x, w_ih_ref[...], (((1,), (1,)), ((), ())),
        preferred_element_type=jnp.float32,
    ) + b_ih_ref[...]
    hb = b_hh_ref[...]
    r = jax.nn.sigmoid(gi[:, :H] + hb[:, :H])
    z = jax.nn.sigmoid(gi[:, H:2 * H] + hb[:, H:2 * H])
    n = jnp.tanh(gi[:, 2 * H:] + r * hb[:, 2 * H:])
    h = (1.0 - z) * n
    logit = jnp.sum(h * w_out_ref[...], axis=1, keepdims=True) + b_out_ref[...]
    out_ref[...] = jax.nn.sigmoid(logit)


def _dense(x, w_ih, b_ih2, b_hh2, w_out, b_out2):
    return pl.pallas_call(
        _dense_body,
        out_shape=jax.ShapeDtypeStruct((B, 1), jnp.float32),
        in_specs=[pl.BlockSpec(memory_space=pltpu.VMEM)] * 6,
        out_specs=pl.BlockSpec(memory_space=pltpu.VMEM),
    )(x, w_ih, b_ih2, b_hh2, w_out, b_out2)


def kernel(indices, labels, table, w_ih, w_hh, b_ih, b_hh, w_out, b_out):
    del w_hh
    idx0 = indices[:, 0, :].reshape(NW, N_CHUNKS, 128)
    x0 = _gather_mean()(idx0, table.T)
    pred = _dense(
        x0,
        w_ih,
        b_ih.reshape(1, 3 * H),
        b_hh.reshape(1, 3 * H),
        w_out,
        b_out.reshape(1, 1),
    )
    return (pred, labels)


# speedup vs baseline: 19.5443x; 19.5443x over previous
import functools

import jax
import jax.numpy as jnp
from jax import lax
from jax.experimental import pallas as pl
from jax.experimental.pallas import tpu as pltpu
from jax.experimental.pallas import tpu_sc as plsc

VOCAB = 1000000
B = 1024
D = 64
H = 128
L = 20

NC = 2
NS = 16
NW = NC * NS
B_PER_W = B // NW
IDX_PER_W = B_PER_W * L
NBUF = 8
BW = 128
T_CLAMP = VOCAB // BW - 2
TAIL_START = VOCAB - 256
TAIL_TH = (T_CLAMP + 1) * BW
D_VECS = D // 16


def _gather_mean_body(idx_hbm, tableT_hbm, tail_hbm, out_hbm,
                      idx_v, rows_v, tail_v, out_v,
                      *sems):
    wid = lax.axis_index("s") * NC + lax.axis_index("c")

    pltpu.sync_copy(idx_hbm.at[wid], idx_v)
    pltpu.sync_copy(tail_hbm, tail_v)

    n_steps = IDX_PER_W // 16

    def load_chunk(g):
        return idx_v[g // 8, pl.ds((g % 8) * 16, 16)]

    def fire(i, k):
        t = jnp.minimum(i // BW, T_CLAMP)
        off = pl.multiple_of(t * BW, BW)
        pltpu.async_copy(
            tableT_hbm.at[:, pl.ds(off, BW)], rows_v.at[k], sems[k])

    chunk0 = load_chunk(0)
    for k in range(NBUF):
        fire(chunk0[k], k)

    zero = jnp.zeros((16,), jnp.float32)
    inv_l = 1.0 / L

    def step(g, carry):
        accs = list(carry[:D_VECS])
        chunk = carry[D_VECS]
        nxt = load_chunk(jnp.minimum(g + 1, n_steps - 1))
        for half in range(2):
            for k in range(NBUF):
                j = g * 16 + half * NBUF + k
                pltpu.make_async_copy(
                    tableT_hbm.at[:, pl.ds(0, BW)], rows_v.at[k], sems[k]
                ).wait()
                i = chunk[half * NBUF + k]
                t = jnp.minimum(i // BW, T_CLAMP)
                lane = jnp.full((16,), i - t * BW, dtype=jnp.int32)
                lane_t = jnp.full(
                    (16,), jnp.clip(i - TAIL_START, 0, 255), jnp.int32)
                m = (i >= TAIL_TH).astype(jnp.float32)
                for g2 in range(D_VECS):
                    rows = jnp.arange(16, dtype=jnp.int32) + 16 * g2
                    main = plsc.load_gather(rows_v.at[k], [rows, lane])
                    tail = plsc.load_gather(tail_v, [rows, lane_t])
                    accs[g2] = accs[g2] + main * (1.0 - m) + tail * m
                if half == 0:
                    fire(chunk[NBUF + k], k)
                else:
                    @pl.when(g < n_steps - 1)
                    def _():
                        fire(nxt[k], k)
                b = j // L
                for g2 in range(D_VECS):
                    out_v[b, pl.ds(16 * g2, 16)] = accs[g2] * inv_l
                keep = 1.0 - ((j % L) == (L - 1)).astype(jnp.float32)
                for g2 in range(D_VECS):
                    accs[g2] = accs[g2] * keep
        return tuple(accs) + (nxt,)

    lax.fori_loop(0, n_steps, step, (zero,) * D_VECS + (chunk0,))

    pltpu.sync_copy(out_v, out_hbm.at[pl.ds(wid * B_PER_W, B_PER_W)])


@functools.cache
def _gather_mean():
    return pl.kernel(
        _gather_mean_body,
        out_type=jax.ShapeDtypeStruct((B, 128), jnp.float32),
        mesh=plsc.VectorSubcoreMesh(core_axis_name="c", subcore_axis_name="s"),
        scratch_types=[
            pltpu.VMEM((8, 128), jnp.int32),
            pltpu.VMEM((NBUF, D, BW), jnp.float32),
            pltpu.VMEM((D, 256), jnp.float32),
            pltpu.VMEM((B_PER_W, 128), jnp.float32),
        ] + [pltpu.SemaphoreType.DMA] * NBUF,
        compiler_params=pltpu.CompilerParams(
            use_tc_tiling_on_sc=True, needs_layout_passes=False),
    )


def _dense_body(x_ref, w_ih_ref, b_ih_ref, b_hh_ref, w_out_ref, b_out_ref,
                out_ref):
    x = x_ref[...][:, :D]
    gi = lax.dot_general(
        x, w_ih_ref[...], (((1,), (1,)), ((), ())),
        preferred_element_type=jnp.float32,
    ) + b_ih_ref[...]
    hb = b_hh_ref[...]
    r = jax.nn.sigmoid(gi[:, :H] + hb[:, :H])
    z = jax.nn.sigmoid(gi[:, H:2 * H] + hb[:, H:2 * H])
    n = jnp.tanh(gi[:, 2 * H:] + r * hb[:, 2 * H:])
    h = (1.0 - z) * n
    logit = jnp.sum(h * w_out_ref[...], axis=1, keepdims=True) + b_out_ref[...]
    out_ref[...] = jax.nn.sigmoid(logit)


def _dense(x, w_ih, b_ih2, b_hh2, w_out, b_out2):
    return pl.pallas_call(
        _dense_body,
        out_shape=jax.ShapeDtypeStruct((B, 1), jnp.float32),
        in_specs=[pl.BlockSpec(memory_space=pltpu.VMEM)] * 6,
        out_specs=pl.BlockSpec(memory_space=pltpu.VMEM),
    )(x, w_ih, b_ih2, b_hh2, w_out, b_out2)


def kernel(indices, labels, table, w_ih, w_hh, b_ih, b_hh, w_out, b_out):
    del w_hh
    idxf = indices[:, 0, :].reshape(NW, IDX_PER_W)
    idx0 = jnp.pad(idxf, ((0, 0), (0, 1024 - IDX_PER_W))).reshape(NW, 8, 128)
    tableT = table.T
    tail = lax.slice(tableT, (0, TAIL_START), (D, VOCAB))
    x0 = _gather_mean()(idx0, tableT, tail)
    pred = _dense(
        x0,
        w_ih,
        b_ih.reshape(1, 3 * H),
        b_hh.reshape(1, 3 * H),
        w_out,
        b_out.reshape(1, 1),
    )
    return (pred, labels)


# speedup vs baseline: 19.6332x; 1.0045x over previous
import functools

import jax
import jax.numpy as jnp
from jax import lax
from jax.experimental import pallas as pl
from jax.experimental.pallas import tpu as pltpu
from jax.experimental.pallas import tpu_sc as plsc

VOCAB = 1000000
B = 1024
D = 64
H = 128
L = 20

NC = 2
NS = 16
NW = NC * NS
B_PER_W = B // NW
IDX_PER_W = B_PER_W * L
NBUF = 8
BW = 128
T_CLAMP = VOCAB // BW - 2
TAIL_START = VOCAB - 256
TAIL_TH = (T_CLAMP + 1) * BW
D_VECS = D // 16


def _gather_mean_body(idx_hbm, tableT_hbm, tail_hbm, out_hbm,
                      idx_v, rows_v, tail_v, out_v,
                      *sems):
    wid = lax.axis_index("s") * NC + lax.axis_index("c")

    pltpu.sync_copy(idx_hbm.at[wid], idx_v)
    pltpu.sync_copy(tail_hbm, tail_v)

    n_steps = IDX_PER_W // 16

    def load_chunk(g):
        return idx_v[g // 8, pl.ds((g % 8) * 16, 16)]

    def fire(i, k):
        t = jnp.minimum(i // BW, T_CLAMP)
        off = pl.multiple_of(t * BW, BW)
        pltpu.async_copy(
            tableT_hbm.at[pl.ds(0, D // 2), pl.ds(off, BW)],
            rows_v.at[k, pl.ds(0, D // 2)], sems[k])
        pltpu.async_copy(
            tableT_hbm.at[pl.ds(D // 2, D // 2), pl.ds(off, BW)],
            rows_v.at[k, pl.ds(D // 2, D // 2)], sems[k])

    chunk0 = load_chunk(0)
    for k in range(NBUF):
        fire(chunk0[k], k)

    zero = jnp.zeros((16,), jnp.float32)
    inv_l = 1.0 / L

    def step(g, carry):
        accs = list(carry[:D_VECS])
        chunk = carry[D_VECS]
        nxt = load_chunk(jnp.minimum(g + 1, n_steps - 1))
        for half in range(2):
            for k in range(NBUF):
                j = g * 16 + half * NBUF + k
                pltpu.make_async_copy(
                    tableT_hbm.at[:, pl.ds(0, BW)], rows_v.at[k], sems[k]
                ).wait()
                i = chunk[half * NBUF + k]
                t = jnp.minimum(i // BW, T_CLAMP)
                lane = jnp.full((16,), i - t * BW, dtype=jnp.int32)
                lane_t = jnp.full(
                    (16,), jnp.clip(i - TAIL_START, 0, 255), jnp.int32)
                m = (i >= TAIL_TH).astype(jnp.float32)
                for g2 in range(D_VECS):
                    rows = jnp.arange(16, dtype=jnp.int32) + 16 * g2
                    main = plsc.load_gather(rows_v.at[k], [rows, lane])
                    tail = plsc.load_gather(tail_v, [rows, lane_t])
                    accs[g2] = accs[g2] + main * (1.0 - m) + tail * m
                if half == 0:
                    fire(chunk[NBUF + k], k)
                else:
                    @pl.when(g < n_steps - 1)
                    def _():
                        fire(nxt[k], k)
                b = j // L
                for g2 in range(D_VECS):
                    out_v[b, pl.ds(16 * g2, 16)] = accs[g2] * inv_l
                keep = 1.0 - ((j % L) == (L - 1)).astype(jnp.float32)
                for g2 in range(D_VECS):
                    accs[g2] = accs[g2] * keep
        return tuple(accs) + (nxt,)

    lax.fori_loop(0, n_steps, step, (zero,) * D_VECS + (chunk0,))

    pltpu.sync_copy(out_v, out_hbm.at[pl.ds(wid * B_PER_W, B_PER_W)])


@functools.cache
def _gather_mean():
    return pl.kernel(
        _gather_mean_body,
        out_type=jax.ShapeDtypeStruct((B, 128), jnp.float32),
        mesh=plsc.VectorSubcoreMesh(core_axis_name="c", subcore_axis_name="s"),
        scratch_types=[
            pltpu.VMEM((8, 128), jnp.int32),
            pltpu.VMEM((NBUF, D, BW), jnp.float32),
            pltpu.VMEM((D, 256), jnp.float32),
            pltpu.VMEM((B_PER_W, 128), jnp.float32),
        ] + [pltpu.SemaphoreType.DMA] * NBUF,
        compiler_params=pltpu.CompilerParams(
            use_tc_tiling_on_sc=True, needs_layout_passes=False),
    )


def _dense_body(x_ref, w_ih_ref, b_ih_ref, b_hh_ref, w_out_ref, b_out_ref,
                out_ref):
    x = x_ref[...][:, :D]
    gi = lax.dot_general(
        x, w_ih_ref[...], (((1,), (1,)), ((), ())),
        preferred_element_type=jnp.float32,
    ) + b_ih_ref[...]
    hb = b_hh_ref[...]
    r = jax.nn.sigmoid(gi[:, :H] + hb[:, :H])
    z = jax.nn.sigmoid(gi[:, H:2 * H] + hb[:, H:2 * H])
    n = jnp.tanh(gi[:, 2 * H:] + r * hb[:, 2 * H:])
    h = (1.0 - z) * n
    logit = jnp.sum(h * w_out_ref[...], axis=1, keepdims=True) + b_out_ref[...]
    out_ref[...] = jax.nn.sigmoid(logit)


def _dense(x, w_ih, b_ih2, b_hh2, w_out, b_out2):
    return pl.pallas_call(
        _dense_body,
        out_shape=jax.ShapeDtypeStruct((B, 1), jnp.float32),
        in_specs=[pl.BlockSpec(memory_space=pltpu.VMEM)] * 6,
        out_specs=pl.BlockSpec(memory_space=pltpu.VMEM),
    )(x, w_ih, b_ih2, b_hh2, w_out, b_out2)


def kernel(indices, labels, table, w_ih, w_hh, b_ih, b_hh, w_out, b_out):
    del w_hh
    idxf = indices[:, 0, :].reshape(NW, IDX_PER_W)
    idx0 = jnp.pad(idxf, ((0, 0), (0, 1024 - IDX_PER_W))).reshape(NW, 8, 128)
    tableT = table.T
    tail = lax.slice(tableT, (0, TAIL_START), (D, VOCAB))
    x0 = _gather_mean()(idx0, tableT, tail)
    pred = _dense(
        x0,
        w_ih,
        b_ih.reshape(1, 3 * H),
        b_hh.reshape(1, 3 * H),
        w_out,
        b_out.reshape(1, 1),
    )
    return (pred, labels)
